# baseline (speedup 1.0000x reference)
import functools

import jax
import jax.numpy as jnp
from jax import lax
from jax.experimental import pallas as pl
from jax.experimental.pallas import tpu as pltpu
from jax.experimental.pallas import tpu_sc as plsc

N = 10000
NP = 10240
E = 160000
D = 256
NW = 32
EPW = E // NW
CPT = D // NW
CPP = 4
CH = 10000
NCH = E // CH
GP = CH // 16
NB = 1024
NGRID = NP // NB
EPS = 1e-5

_SC_PARAMS = pltpu.CompilerParams(needs_layout_passes=False)


@functools.cache
def _mesh():
    return plsc.VectorSubcoreMesh(core_axis_name="c", subcore_axis_name="s",
                                  num_cores=2, num_subcores=16)


def _worker_id():
    return lax.axis_index("s") * 2 + lax.axis_index("c")


def _deg_body(dst_hbm, part_hbm, dbuf, acc):
    w = _worker_id()

    @pl.loop(0, NP // 16)
    def _zero(i):
        acc[pl.ds(i * 16, 16)] = jnp.zeros((16,), jnp.float32)

    pltpu.sync_copy(dst_hbm.at[pl.ds(w * EPW, EPW)], dbuf.at[pl.ds(0, EPW)])
    ones = jnp.ones((16,), jnp.float32)

    @plsc.parallel_loop(0, (EPW // 16) * 16, step=16, unroll=4)
    def _count(b):
        t = dbuf[pl.ds(b, 16)]
        plsc.addupdate_scatter(acc, [t], ones)

    lane = lax.iota(jnp.int32, 16)
    m = lane < (EPW % 16)
    t = dbuf[pl.ds((EPW // 16) * 16, 16)]
    t = jnp.where(m, t, 0)
    plsc.addupdate_scatter(acc, [t], ones, mask=m)

    pltpu.sync_copy(acc, part_hbm.at[w])


@functools.cache
def _deg_kernel():
    return pl.kernel(
        _deg_body,
        out_type=jax.ShapeDtypeStruct((NW, NP), jnp.float32),
        mesh=_mesh(),
        compiler_params=_SC_PARAMS,
        scratch_types=[
            pltpu.VMEM((EPW + 16, ), jnp.int32),
            pltpu.VMEM((NP,), jnp.float32),
        ],
    )


def _agg_body(ht_hbm, src_hbm, dst_hbm, acc_hbm,
              sA, sB, tA, tB, h0, h1, h2, h3, a0, a1, a2, a3,
              sem0, sem1, sem2, sem3):
    w = _worker_id()
    hbufs = [h0, h1, h2, h3]
    abufs = [a0, a1, a2, a3]

    def start(ci):
        bs, bt = (sA, tA) if ci % 2 == 0 else (sB, tB)
        ss, st = (sem0, sem1) if ci % 2 == 0 else (sem2, sem3)
        return (pltpu.async_copy(src_hbm.at[pl.ds(ci * CH, CH)], bs, ss),
                pltpu.async_copy(dst_hbm.at[pl.ds(ci * CH, CH)], bt, st))

    for p in range(CPT // CPP):
        base_col = w * CPT + p * CPP
        for j in range(CPP):
            pltpu.sync_copy(ht_hbm.at[base_col + j], hbufs[j])
            pltpu.sync_copy(ht_hbm.at[base_col + j], abufs[j])
        pend = start(0)
        for ci in range(NCH):
            cs, ct = (sA, tA) if ci % 2 == 0 else (sB, tB)
            pend[0].wait()
            pend[1].wait()
            if ci + 1 < NCH:
                pend_next = start(ci + 1)

            @plsc.parallel_loop(0, CH, step=16, unroll=8)
            def _groups(b, cs=cs, ct=ct):
                si = cs[pl.ds(b, 16)]
                ti = ct[pl.ds(b, 16)]
                vs = [plsc.load_gather(hbufs[j], [si]) for j in range(CPP)]
                for j in range(CPP):
                    plsc.addupdate_scatter(abufs[j], [ti], vs[j])

            if ci + 1 < NCH:
                pend = pend_next
        for j in range(CPP):
            pltpu.sync_copy(abufs[j], acc_hbm.at[base_col + j])


@functools.cache
def _agg_kernel():
    return pl.kernel(
        _agg_body,
        out_type=jax.ShapeDtypeStruct((D, NP), jnp.float32),
        mesh=_mesh(),
        compiler_params=_SC_PARAMS,
        scratch_types=[
        pltpu.VMEM((CH,), jnp.int32),
        pltpu.VMEM((CH,), jnp.int32),
        pltpu.VMEM((CH,), jnp.int32),
        pltpu.VMEM((CH,), jnp.int32),
        pltpu.VMEM((NP,), jnp.float32),
        pltpu.VMEM((NP,), jnp.float32),
        pltpu.VMEM((NP,), jnp.float32),
        pltpu.VMEM((NP,), jnp.float32),
        pltpu.VMEM((NP,), jnp.float32),
        pltpu.VMEM((NP,), jnp.float32),
        pltpu.VMEM((NP,), jnp.float32),
        pltpu.VMEM((NP,), jnp.float32),
            pltpu.SemaphoreType.DMA,
            pltpu.SemaphoreType.DMA,
            pltpu.SemaphoreType.DMA,
            pltpu.SemaphoreType.DMA,
        ],
    )


def _prep_body(part_ref, x_ref, w1_ref, ht_ref, dinv_ref):
    deg = jnp.sum(part_ref[...], axis=0, keepdims=True) + 1.0
    dinv = lax.rsqrt(deg)
    dinv_ref[...] = dinv
    h = jnp.dot(x_ref[...], w1_ref[...],
                preferred_element_type=jnp.float32,
                precision=lax.Precision.HIGHEST)
    ht_ref[...] = h.T * dinv


_prep_kernel = pl.pallas_call(
    _prep_body,
    grid=(NGRID,),
    in_specs=[
        pl.BlockSpec((NW, NB), lambda i: (0, i)),
        pl.BlockSpec((NB, D), lambda i: (i, 0)),
        pl.BlockSpec((D, D), lambda i: (0, 0)),
    ],
    out_specs=[
        pl.BlockSpec((D, NB), lambda i: (0, i)),
        pl.BlockSpec((1, NB), lambda i: (0, i)),
    ],
    out_shape=[
        jax.ShapeDtypeStruct((D, NP), jnp.float32),
        jax.ShapeDtypeStruct((1, NP), jnp.float32),
    ],
)


def _stats_body(acc_ref, dinv_ref, s_ref):
    i = pl.program_id(0)
    t = acc_ref[...] * dinv_ref[...]
    s1 = jnp.sum(t, axis=1, keepdims=True)
    s2 = jnp.sum(t * t, axis=1, keepdims=True)
    sb = jnp.concatenate([s1, s2], axis=1)

    @pl.when(i == 0)
    def _init():
        s_ref[...] = sb

    @pl.when(i > 0)
    def _accum():
        s_ref[...] += sb


_stats_kernel = pl.pallas_call(
    _stats_body,
    grid=(NGRID,),
    in_specs=[
        pl.BlockSpec((D, NB), lambda i: (0, i)),
        pl.BlockSpec((1, NB), lambda i: (0, i)),
    ],
    out_specs=pl.BlockSpec((D, 2), lambda i: (0, 0)),
    out_shape=jax.ShapeDtypeStruct((D, 2), jnp.float32),
)


def _bn_coeffs(s_ref, g_ref, be_ref):
    m = s_ref[:, 0:1] * (1.0 / N)
    v = s_ref[:, 1:2] * (1.0 / N) - m * m
    alpha = g_ref[...] * lax.rsqrt(v + EPS)
    beta = be_ref[...] - alpha * m
    return alpha, beta


def _mid_body(acc_ref, dinv_ref, s_ref, g_ref, be_ref, w2t_ref, mask_ref,
              ht2_ref):
    alpha, beta = _bn_coeffs(s_ref, g_ref, be_ref)
    pre = acc_ref[...] * dinv_ref[...]
    r = jnp.maximum(alpha * pre + beta, 0.0) * mask_ref[...]
    ht2_ref[...] = jnp.dot(w2t_ref[...], r,
                           preferred_element_type=jnp.float32,
                           precision=lax.Precision.HIGHEST) * dinv_ref[...]


_mid_kernel = pl.pallas_call(
    _mid_body,
    grid=(NGRID,),
    in_specs=[
        pl.BlockSpec((D, NB), lambda i: (0, i)),
        pl.BlockSpec((1, NB), lambda i: (0, i)),
        pl.BlockSpec((D, 2), lambda i: (0, 0)),
        pl.BlockSpec((D, 1), lambda i: (0, 0)),
        pl.BlockSpec((D, 1), lambda i: (0, 0)),
        pl.BlockSpec((D, D), lambda i: (0, 0)),
        pl.BlockSpec((1, NB), lambda i: (0, i)),
    ],
    out_specs=pl.BlockSpec((D, NB), lambda i: (0, i)),
    out_shape=jax.ShapeDtypeStruct((D, NP), jnp.float32),
)


def _final_body(acc_ref, dinv_ref, s_ref, g_ref, be_ref, x_ref, out_ref):
    alpha, beta = _bn_coeffs(s_ref, g_ref, be_ref)
    pre = acc_ref[...] * dinv_ref[...]
    r = jnp.maximum(alpha * pre + beta, 0.0)
    out_ref[...] = r.T + x_ref[...]


_final_kernel = pl.pallas_call(
    _final_body,
    grid=(NGRID,),
    in_specs=[
        pl.BlockSpec((D, NB), lambda i: (0, i)),
        pl.BlockSpec((1, NB), lambda i: (0, i)),
        pl.BlockSpec((D, 2), lambda i: (0, 0)),
        pl.BlockSpec((D, 1), lambda i: (0, 0)),
        pl.BlockSpec((D, 1), lambda i: (0, 0)),
        pl.BlockSpec((NB, D), lambda i: (i, 0)),
    ],
    out_specs=pl.BlockSpec((NB, D), lambda i: (i, 0)),
    out_shape=jax.ShapeDtypeStruct((NP, D), jnp.float32),
)


@jax.jit
def kernel(x, edge_index, W1, b1, g1, be1, W2, b2, g2, be2):
    del b1, b2
    xp = jnp.pad(x, ((0, NP - N), (0, 0)))
    src = edge_index[0]
    dst = edge_index[1]
    mask = (jnp.arange(NP, dtype=jnp.int32) < N).astype(jnp.float32)
    mask = mask.reshape(1, NP)

    part = _deg_kernel()(dst)
    ht1, dinv = _prep_kernel(part, xp, W1)
    acc1 = _agg_kernel()(ht1, src, dst)
    s1 = _stats_kernel(acc1, dinv)
    ht2 = _mid_kernel(acc1, dinv, s1, g1.reshape(D, 1), be1.reshape(D, 1),
                      W2.T, mask)
    acc2 = _agg_kernel()(ht2, src, dst)
    s2 = _stats_kernel(acc2, dinv)
    outp = _final_kernel(acc2, dinv, s2, g2.reshape(D, 1), be2.reshape(D, 1),
                         xp)
    return outp[:N]


# speedup vs baseline: 1.0321x; 1.0321x over previous
import functools

import jax
import jax.numpy as jnp
from jax import lax
from jax.experimental import pallas as pl
from jax.experimental.pallas import tpu as pltpu
from jax.experimental.pallas import tpu_sc as plsc

N = 10000
NP = 10240
E = 160000
D = 256
NW = 32
EPW = E // NW
CPT = D // NW
CPP = 4
CH = 10000
NCH = E // CH
GP = CH // 16
NB = 1024
NGRID = NP // NB
EPS = 1e-5

_SC_PARAMS = pltpu.CompilerParams(needs_layout_passes=False)


@functools.cache
def _mesh():
    return plsc.VectorSubcoreMesh(core_axis_name="c", subcore_axis_name="s",
                                  num_cores=2, num_subcores=16)


def _worker_id():
    return lax.axis_index("s") * 2 + lax.axis_index("c")


def _deg_body(dst_hbm, part_hbm, dbuf, acc):
    w = _worker_id()

    @pl.loop(0, NP // 16)
    def _zero(i):
        acc[pl.ds(i * 16, 16)] = jnp.zeros((16,), jnp.float32)

    pltpu.sync_copy(dst_hbm.at[pl.ds(w * EPW, EPW)], dbuf.at[pl.ds(0, EPW)])
    ones = jnp.ones((16,), jnp.float32)

    @plsc.parallel_loop(0, (EPW // 16) * 16, step=16, unroll=4)
    def _count(b):
        t = dbuf[pl.ds(b, 16)]
        plsc.addupdate_scatter(acc, [t], ones)

    lane = lax.iota(jnp.int32, 16)
    m = lane < (EPW % 16)
    t = dbuf[pl.ds((EPW // 16) * 16, 16)]
    t = jnp.where(m, t, 0)
    plsc.addupdate_scatter(acc, [t], ones, mask=m)

    pltpu.sync_copy(acc, part_hbm.at[w])


@functools.cache
def _deg_kernel():
    return pl.kernel(
        _deg_body,
        out_type=jax.ShapeDtypeStruct((NW, NP), jnp.float32),
        mesh=_mesh(),
        compiler_params=_SC_PARAMS,
        scratch_types=[
            pltpu.VMEM((EPW + 16, ), jnp.int32),
            pltpu.VMEM((NP,), jnp.float32),
        ],
    )


def _agg_body(ht_hbm, src_hbm, dst_hbm, acc_hbm,
              sA, sB, tA, tB, h0, h1, h2, h3, a0, a1, a2, a3,
              sem0, sem1, sem2, sem3):
    w = _worker_id()
    hbufs = [h0, h1, h2, h3]
    abufs = [a0, a1, a2, a3]

    def start(ci):
        bs, bt = (sA, tA) if ci % 2 == 0 else (sB, tB)
        ss, st = (sem0, sem1) if ci % 2 == 0 else (sem2, sem3)
        return (pltpu.async_copy(src_hbm.at[pl.ds(ci * CH, CH)], bs, ss),
                pltpu.async_copy(dst_hbm.at[pl.ds(ci * CH, CH)], bt, st))

    for p in range(CPT // CPP):
        base_col = w * CPT + p * CPP
        for j in range(CPP):
            pltpu.sync_copy(ht_hbm.at[base_col + j], hbufs[j])
            pltpu.sync_copy(ht_hbm.at[base_col + j], abufs[j])
        pend = start(0)
        for ci in range(NCH):
            cs, ct = (sA, tA) if ci % 2 == 0 else (sB, tB)
            pend[0].wait()
            pend[1].wait()
            if ci + 1 < NCH:
                pend_next = start(ci + 1)

            @plsc.parallel_loop(0, CH, step=16, unroll=4)
            def _groups(b, cs=cs, ct=ct):
                si = cs[pl.ds(b, 16)]
                ti = ct[pl.ds(b, 16)]
                vs = [plsc.load_gather(hbufs[j], [si]) for j in range(CPP)]
                for j in range(CPP):
                    plsc.addupdate_scatter(abufs[j], [ti], vs[j])

            if ci + 1 < NCH:
                pend = pend_next
        for j in range(CPP):
            pltpu.sync_copy(abufs[j], acc_hbm.at[base_col + j])


@functools.cache
def _agg_kernel():
    return pl.kernel(
        _agg_body,
        out_type=jax.ShapeDtypeStruct((D, NP), jnp.float32),
        mesh=_mesh(),
        compiler_params=_SC_PARAMS,
        scratch_types=[
        pltpu.VMEM((CH,), jnp.int32),
        pltpu.VMEM((CH,), jnp.int32),
        pltpu.VMEM((CH,), jnp.int32),
        pltpu.VMEM((CH,), jnp.int32),
        pltpu.VMEM((NP,), jnp.float32),
        pltpu.VMEM((NP,), jnp.float32),
        pltpu.VMEM((NP,), jnp.float32),
        pltpu.VMEM((NP,), jnp.float32),
        pltpu.VMEM((NP,), jnp.float32),
        pltpu.VMEM((NP,), jnp.float32),
        pltpu.VMEM((NP,), jnp.float32),
        pltpu.VMEM((NP,), jnp.float32),
            pltpu.SemaphoreType.DMA,
            pltpu.SemaphoreType.DMA,
            pltpu.SemaphoreType.DMA,
            pltpu.SemaphoreType.DMA,
        ],
    )


def _prep_body(part_ref, x_ref, w1_ref, ht_ref, dinv_ref):
    deg = jnp.sum(part_ref[...], axis=0, keepdims=True) + 1.0
    dinv = lax.rsqrt(deg)
    dinv_ref[...] = dinv
    h = jnp.dot(x_ref[...], w1_ref[...],
                preferred_element_type=jnp.float32,
                precision=lax.Precision.HIGHEST)
    ht_ref[...] = h.T * dinv


_prep_kernel = pl.pallas_call(
    _prep_body,
    grid=(NGRID,),
    in_specs=[
        pl.BlockSpec((NW, NB), lambda i: (0, i)),
        pl.BlockSpec((NB, D), lambda i: (i, 0)),
        pl.BlockSpec((D, D), lambda i: (0, 0)),
    ],
    out_specs=[
        pl.BlockSpec((D, NB), lambda i: (0, i)),
        pl.BlockSpec((1, NB), lambda i: (0, i)),
    ],
    out_shape=[
        jax.ShapeDtypeStruct((D, NP), jnp.float32),
        jax.ShapeDtypeStruct((1, NP), jnp.float32),
    ],
)


def _stats_body(acc_ref, dinv_ref, s_ref):
    i = pl.program_id(0)
    t = acc_ref[...] * dinv_ref[...]
    s1 = jnp.sum(t, axis=1, keepdims=True)
    s2 = jnp.sum(t * t, axis=1, keepdims=True)
    sb = jnp.concatenate([s1, s2], axis=1)

    @pl.when(i == 0)
    def _init():
        s_ref[...] = sb

    @pl.when(i > 0)
    def _accum():
        s_ref[...] += sb


_stats_kernel = pl.pallas_call(
    _stats_body,
    grid=(NGRID,),
    in_specs=[
        pl.BlockSpec((D, NB), lambda i: (0, i)),
        pl.BlockSpec((1, NB), lambda i: (0, i)),
    ],
    out_specs=pl.BlockSpec((D, 2), lambda i: (0, 0)),
    out_shape=jax.ShapeDtypeStruct((D, 2), jnp.float32),
)


def _bn_coeffs(s_ref, g_ref, be_ref):
    m = s_ref[:, 0:1] * (1.0 / N)
    v = s_ref[:, 1:2] * (1.0 / N) - m * m
    alpha = g_ref[...] * lax.rsqrt(v + EPS)
    beta = be_ref[...] - alpha * m
    return alpha, beta


def _mid_body(acc_ref, dinv_ref, s_ref, g_ref, be_ref, w2t_ref, mask_ref,
              ht2_ref):
    alpha, beta = _bn_coeffs(s_ref, g_ref, be_ref)
    pre = acc_ref[...] * dinv_ref[...]
    r = jnp.maximum(alpha * pre + beta, 0.0) * mask_ref[...]
    ht2_ref[...] = jnp.dot(w2t_ref[...], r,
                           preferred_element_type=jnp.float32,
                           precision=lax.Precision.HIGHEST) * dinv_ref[...]


_mid_kernel = pl.pallas_call(
    _mid_body,
    grid=(NGRID,),
    in_specs=[
        pl.BlockSpec((D, NB), lambda i: (0, i)),
        pl.BlockSpec((1, NB), lambda i: (0, i)),
        pl.BlockSpec((D, 2), lambda i: (0, 0)),
        pl.BlockSpec((D, 1), lambda i: (0, 0)),
        pl.BlockSpec((D, 1), lambda i: (0, 0)),
        pl.BlockSpec((D, D), lambda i: (0, 0)),
        pl.BlockSpec((1, NB), lambda i: (0, i)),
    ],
    out_specs=pl.BlockSpec((D, NB), lambda i: (0, i)),
    out_shape=jax.ShapeDtypeStruct((D, NP), jnp.float32),
)


def _final_body(acc_ref, dinv_ref, s_ref, g_ref, be_ref, x_ref, out_ref):
    alpha, beta = _bn_coeffs(s_ref, g_ref, be_ref)
    pre = acc_ref[...] * dinv_ref[...]
    r = jnp.maximum(alpha * pre + beta, 0.0)
    out_ref[...] = r.T + x_ref[...]


_final_kernel = pl.pallas_call(
    _final_body,
    grid=(NGRID,),
    in_specs=[
        pl.BlockSpec((D, NB), lambda i: (0, i)),
        pl.BlockSpec((1, NB), lambda i: (0, i)),
        pl.BlockSpec((D, 2), lambda i: (0, 0)),
        pl.BlockSpec((D, 1), lambda i: (0, 0)),
        pl.BlockSpec((D, 1), lambda i: (0, 0)),
        pl.BlockSpec((NB, D), lambda i: (i, 0)),
    ],
    out_specs=pl.BlockSpec((NB, D), lambda i: (i, 0)),
    out_shape=jax.ShapeDtypeStruct((NP, D), jnp.float32),
)


@jax.jit
def kernel(x, edge_index, W1, b1, g1, be1, W2, b2, g2, be2):
    del b1, b2
    xp = jnp.pad(x, ((0, NP - N), (0, 0)))
    src = edge_index[0]
    dst = edge_index[1]
    mask = (jnp.arange(NP, dtype=jnp.int32) < N).astype(jnp.float32)
    mask = mask.reshape(1, NP)

    part = _deg_kernel()(dst)
    ht1, dinv = _prep_kernel(part, xp, W1)
    acc1 = _agg_kernel()(ht1, src, dst)
    s1 = _stats_kernel(acc1, dinv)
    ht2 = _mid_kernel(acc1, dinv, s1, g1.reshape(D, 1), be1.reshape(D, 1),
                      W2.T, mask)
    acc2 = _agg_kernel()(ht2, src, dst)
    s2 = _stats_kernel(acc2, dinv)
    outp = _final_kernel(acc2, dinv, s2, g2.reshape(D, 1), be2.reshape(D, 1),
                         xp)
    return outp[:N]


# speedup vs baseline: 1.0891x; 1.0552x over previous
import functools

import jax
import jax.numpy as jnp
from jax import lax
from jax.experimental import pallas as pl
from jax.experimental.pallas import tpu as pltpu
from jax.experimental.pallas import tpu_sc as plsc

N = 10000
NP = 10240
E = 160000
D = 256
NW = 32
EPW = E // NW
CPT = D // NW
CPP = 4
CH = 10000
NCH = E // CH
GP = CH // 16
NB = 1024
NGRID = NP // NB
EPS = 1e-5

_SC_PARAMS = pltpu.CompilerParams(needs_layout_passes=False)


@functools.cache
def _mesh():
    return plsc.VectorSubcoreMesh(core_axis_name="c", subcore_axis_name="s",
                                  num_cores=2, num_subcores=16)


def _worker_id():
    return lax.axis_index("s") * 2 + lax.axis_index("c")


def _deg_body(src_hbm, dst_hbm, part_hbm, pk_hbm, sbuf, dbuf, pbuf, acc):
    w = _worker_id()

    @pl.loop(0, NP // 16)
    def _zero(i):
        acc[pl.ds(i * 16, 16)] = jnp.zeros((16,), jnp.float32)

    pltpu.sync_copy(src_hbm.at[pl.ds(w * EPW, EPW)], sbuf.at[pl.ds(0, EPW)])
    pltpu.sync_copy(dst_hbm.at[pl.ds(w * EPW, EPW)], dbuf.at[pl.ds(0, EPW)])
    ones = jnp.ones((16,), jnp.float32)

    @plsc.parallel_loop(0, (EPW // 16) * 16, step=16, unroll=4)
    def _count(b):
        t = dbuf[pl.ds(b, 16)]
        plsc.addupdate_scatter(acc, [t], ones)
        pbuf[pl.ds(b, 16)] = jnp.left_shift(t, 14) + sbuf[pl.ds(b, 16)]

    lane = lax.iota(jnp.int32, 16)
    m = lane < (EPW % 16)
    b0 = (EPW // 16) * 16
    t = dbuf[pl.ds(b0, 16)]
    t = jnp.where(m, t, 0)
    s = jnp.where(m, sbuf[pl.ds(b0, 16)], 0)
    plsc.addupdate_scatter(acc, [t], ones, mask=m)
    pbuf[pl.ds(b0, 16)] = jnp.left_shift(t, 14) + s

    pltpu.sync_copy(acc, part_hbm.at[w])
    pltpu.sync_copy(pbuf.at[pl.ds(0, EPW)], pk_hbm.at[pl.ds(w * EPW, EPW)])


@functools.cache
def _deg_kernel():
    return pl.kernel(
        _deg_body,
        out_type=[jax.ShapeDtypeStruct((NW, NP), jnp.float32),
                  jax.ShapeDtypeStruct((E,), jnp.int32)],
        mesh=_mesh(),
        compiler_params=_SC_PARAMS,
        scratch_types=[
            pltpu.VMEM((EPW + 16, ), jnp.int32),
            pltpu.VMEM((EPW + 16, ), jnp.int32),
            pltpu.VMEM((EPW + 16, ), jnp.int32),
            pltpu.VMEM((NP,), jnp.float32),
        ],
    )


def _agg_body(ht_hbm, pk_hbm, dinv_hbm, acc_hbm, stat_hbm,
              pA, pB, h0, h1, h2, h3, a0, a1, a2, a3, dv, sbuf,
              sem0, sem1):
    w = _worker_id()
    hbufs = [h0, h1, h2, h3]
    abufs = [a0, a1, a2, a3]
    lane = lax.iota(jnp.int32, 16)
    zeros16 = jnp.zeros((16,), jnp.float32)

    pltpu.sync_copy(dinv_hbm.at[0], dv)

    def start(ci):
        bp = pA if ci % 2 == 0 else pB
        sp = sem0 if ci % 2 == 0 else sem1
        return pltpu.async_copy(pk_hbm.at[pl.ds(ci * CH, CH)], bp, sp)

    for p in range(CPT // CPP):
        base_col = w * CPT + p * CPP
        for j in range(CPP):
            pltpu.sync_copy(ht_hbm.at[base_col + j], hbufs[j])
            pltpu.sync_copy(ht_hbm.at[base_col + j], abufs[j])
        pend = start(0)
        for ci in range(NCH):
            cp = pA if ci % 2 == 0 else pB
            pend.wait()
            if ci + 1 < NCH:
                pend_next = start(ci + 1)

            @plsc.parallel_loop(0, CH, step=16, unroll=4)
            def _groups(b, cp=cp):
                pk = cp[pl.ds(b, 16)]
                si = jnp.bitwise_and(pk, 16383)
                ti = jnp.right_shift(pk, 14)
                vs = [plsc.load_gather(hbufs[j], [si]) for j in range(CPP)]
                for j in range(CPP):
                    plsc.addupdate_scatter(abufs[j], [ti], vs[j])

            if ci + 1 < NCH:
                pend = pend_next
        for j in range(CPP):
            pltpu.sync_copy(abufs[j], acc_hbm.at[base_col + j])
            abuf = abufs[j]

            @pl.loop(0, NP // 16, init_carry=(zeros16, zeros16), unroll=4)
            def _sums(i, c, abuf=abuf):
                s1v, s2v = c
                t = abuf[pl.ds(i * 16, 16)] * dv[pl.ds(i * 16, 16)]
                return (s1v + t, s2v + t * t)

            s1v, s2v = _sums
            s1 = jnp.sum(s1v)
            s2 = jnp.sum(s2v)
            sbuf[...] = jnp.where(lane == 0, s1,
                                  jnp.where(lane == 1, s2, 0.0))
            pltpu.sync_copy(sbuf, stat_hbm.at[base_col + j])


@functools.cache
def _agg_kernel():
    return pl.kernel(
        _agg_body,
        out_type=[jax.ShapeDtypeStruct((D, NP), jnp.float32),
                  jax.ShapeDtypeStruct((D, 16), jnp.float32)],
        mesh=_mesh(),
        compiler_params=_SC_PARAMS,
        scratch_types=[
            pltpu.VMEM((CH,), jnp.int32),
            pltpu.VMEM((CH,), jnp.int32),
            pltpu.VMEM((NP,), jnp.float32),
            pltpu.VMEM((NP,), jnp.float32),
            pltpu.VMEM((NP,), jnp.float32),
            pltpu.VMEM((NP,), jnp.float32),
            pltpu.VMEM((NP,), jnp.float32),
            pltpu.VMEM((NP,), jnp.float32),
            pltpu.VMEM((NP,), jnp.float32),
            pltpu.VMEM((NP,), jnp.float32),
            pltpu.VMEM((NP,), jnp.float32),
            pltpu.VMEM((16,), jnp.float32),
            pltpu.SemaphoreType.DMA,
            pltpu.SemaphoreType.DMA,
        ],
    )


def _prep_body(part_ref, x_ref, w1_ref, ht_ref, dinv_ref):
    deg = jnp.sum(part_ref[...], axis=0, keepdims=True) + 1.0
    dinv = lax.rsqrt(deg)
    dinv_ref[...] = dinv
    h = jnp.dot(x_ref[...], w1_ref[...],
                preferred_element_type=jnp.float32,
                precision=lax.Precision.HIGHEST)
    ht_ref[...] = h.T * dinv


_prep_kernel = pl.pallas_call(
    _prep_body,
    grid=(NGRID,),
    in_specs=[
        pl.BlockSpec((NW, NB), lambda i: (0, i)),
        pl.BlockSpec((NB, D), lambda i: (i, 0)),
        pl.BlockSpec((D, D), lambda i: (0, 0)),
    ],
    out_specs=[
        pl.BlockSpec((D, NB), lambda i: (0, i)),
        pl.BlockSpec((1, NB), lambda i: (0, i)),
    ],
    out_shape=[
        jax.ShapeDtypeStruct((D, NP), jnp.float32),
        jax.ShapeDtypeStruct((1, NP), jnp.float32),
    ],
)


def _bn_coeffs(s_ref, g_ref, be_ref):
    m = s_ref[:, 0:1] * (1.0 / N)
    v = s_ref[:, 1:2] * (1.0 / N) - m * m
    alpha = g_ref[...] * lax.rsqrt(v + EPS)
    beta = be_ref[...] - alpha * m
    return alpha, beta


def _mid_body(acc_ref, dinv_ref, s_ref, g_ref, be_ref, w2t_ref, mask_ref,
              ht2_ref):
    alpha, beta = _bn_coeffs(s_ref, g_ref, be_ref)
    pre = acc_ref[...] * dinv_ref[...]
    r = jnp.maximum(alpha * pre + beta, 0.0) * mask_ref[...]
    ht2_ref[...] = jnp.dot(w2t_ref[...], r,
                           preferred_element_type=jnp.float32,
                           precision=lax.Precision.HIGHEST) * dinv_ref[...]


_mid_kernel = pl.pallas_call(
    _mid_body,
    grid=(NGRID,),
    in_specs=[
        pl.BlockSpec((D, NB), lambda i: (0, i)),
        pl.BlockSpec((1, NB), lambda i: (0, i)),
        pl.BlockSpec((D, 16), lambda i: (0, 0)),
        pl.BlockSpec((D, 1), lambda i: (0, 0)),
        pl.BlockSpec((D, 1), lambda i: (0, 0)),
        pl.BlockSpec((D, D), lambda i: (0, 0)),
        pl.BlockSpec((1, NB), lambda i: (0, i)),
    ],
    out_specs=pl.BlockSpec((D, NB), lambda i: (0, i)),
    out_shape=jax.ShapeDtypeStruct((D, NP), jnp.float32),
)


def _final_body(acc_ref, dinv_ref, s_ref, g_ref, be_ref, x_ref, out_ref):
    alpha, beta = _bn_coeffs(s_ref, g_ref, be_ref)
    pre = acc_ref[...] * dinv_ref[...]
    r = jnp.maximum(alpha * pre + beta, 0.0)
    out_ref[...] = r.T + x_ref[...]


_final_kernel = pl.pallas_call(
    _final_body,
    grid=(NGRID,),
    in_specs=[
        pl.BlockSpec((D, NB), lambda i: (0, i)),
        pl.BlockSpec((1, NB), lambda i: (0, i)),
        pl.BlockSpec((D, 16), lambda i: (0, 0)),
        pl.BlockSpec((D, 1), lambda i: (0, 0)),
        pl.BlockSpec((D, 1), lambda i: (0, 0)),
        pl.BlockSpec((NB, D), lambda i: (i, 0)),
    ],
    out_specs=pl.BlockSpec((NB, D), lambda i: (i, 0)),
    out_shape=jax.ShapeDtypeStruct((NP, D), jnp.float32),
)


@jax.jit
def kernel(x, edge_index, W1, b1, g1, be1, W2, b2, g2, be2):
    del b1, b2
    xp = jnp.pad(x, ((0, NP - N), (0, 0)))
    src = edge_index[0]
    dst = edge_index[1]
    mask = (jnp.arange(NP, dtype=jnp.int32) < N).astype(jnp.float32)
    mask = mask.reshape(1, NP)

    part, packed = _deg_kernel()(src, dst)
    ht1, dinv = _prep_kernel(part, xp, W1)
    acc1, s1 = _agg_kernel()(ht1, packed, dinv)
    ht2 = _mid_kernel(acc1, dinv, s1, g1.reshape(D, 1), be1.reshape(D, 1),
                      W2.T, mask)
    acc2, s2 = _agg_kernel()(ht2, packed, dinv)
    outp = _final_kernel(acc2, dinv, s2, g2.reshape(D, 1), be2.reshape(D, 1),
                         xp)
    return outp[:N]


# speedup vs baseline: 1.1382x; 1.0451x over previous
import functools

import jax
import jax.numpy as jnp
from jax import lax
from jax.experimental import pallas as pl
from jax.experimental.pallas import tpu as pltpu
from jax.experimental.pallas import tpu_sc as plsc

N = 10000
NP = 10240
E = 160000
D = 256
NW = 32
EPW = E // NW
CPT = D // NW
CPP = 4
CH = 10000
NCH = E // CH
GP = CH // 16
NB = 1024
NGRID = NP // NB
EPS = 1e-5

_SC_PARAMS = pltpu.CompilerParams(needs_layout_passes=False)


@functools.cache
def _mesh():
    return plsc.VectorSubcoreMesh(core_axis_name="c", subcore_axis_name="s",
                                  num_cores=2, num_subcores=16)


def _worker_id():
    return lax.axis_index("s") * 2 + lax.axis_index("c")


def _deg_body(src_hbm, dst_hbm, part_hbm, pk_hbm, sbuf, dbuf, pbuf, acc):
    w = _worker_id()

    @pl.loop(0, NP // 16)
    def _zero(i):
        acc[pl.ds(i * 16, 16)] = jnp.zeros((16,), jnp.float32)

    pltpu.sync_copy(src_hbm.at[pl.ds(w * EPW, EPW)], sbuf.at[pl.ds(0, EPW)])
    pltpu.sync_copy(dst_hbm.at[pl.ds(w * EPW, EPW)], dbuf.at[pl.ds(0, EPW)])
    ones = jnp.ones((16,), jnp.float32)

    @plsc.parallel_loop(0, (EPW // 16) * 16, step=16, unroll=4)
    def _count(b):
        t = dbuf[pl.ds(b, 16)]
        plsc.addupdate_scatter(acc, [t], ones)
        pbuf[pl.ds(b, 16)] = jnp.left_shift(t, 14) + sbuf[pl.ds(b, 16)]

    lane = lax.iota(jnp.int32, 16)
    m = lane < (EPW % 16)
    b0 = (EPW // 16) * 16
    t = dbuf[pl.ds(b0, 16)]
    t = jnp.where(m, t, 0)
    s = jnp.where(m, sbuf[pl.ds(b0, 16)], 0)
    plsc.addupdate_scatter(acc, [t], ones, mask=m)
    pbuf[pl.ds(b0, 16)] = jnp.left_shift(t, 14) + s

    pltpu.sync_copy(acc, part_hbm.at[w])
    pltpu.sync_copy(pbuf.at[pl.ds(0, EPW)], pk_hbm.at[pl.ds(w * EPW, EPW)])


@functools.cache
def _deg_kernel():
    return pl.kernel(
        _deg_body,
        out_type=[jax.ShapeDtypeStruct((NW, NP), jnp.float32),
                  jax.ShapeDtypeStruct((E,), jnp.int32)],
        mesh=_mesh(),
        compiler_params=_SC_PARAMS,
        scratch_types=[
            pltpu.VMEM((EPW + 16, ), jnp.int32),
            pltpu.VMEM((EPW + 16, ), jnp.int32),
            pltpu.VMEM((EPW + 16, ), jnp.int32),
            pltpu.VMEM((NP,), jnp.float32),
        ],
    )


def _agg_body(ht_hbm, pk_hbm, dinv_hbm, acc_hbm, stat_hbm,
              pA, pB, h0, h1, h2, h3, a0, a1, a2, a3, dv, sbuf,
              sem0, sem1):
    w = _worker_id()
    hbufs = [h0, h1, h2, h3]
    abufs = [a0, a1, a2, a3]
    lane = lax.iota(jnp.int32, 16)
    zeros16 = jnp.zeros((16,), jnp.float32)

    pltpu.sync_copy(dinv_hbm.at[0], dv)

    def start(ci):
        bp = pA if ci % 2 == 0 else pB
        sp = sem0 if ci % 2 == 0 else sem1
        return pltpu.async_copy(pk_hbm.at[pl.ds(ci * CH, CH)], bp, sp)

    wb = []
    for p in range(CPT // CPP):
        base_col = w * CPT + p * CPP
        for d in wb:
            d.wait()
        wb = []
        inits = []
        for j in range(CPP):
            inits.append(
                pltpu.async_copy(ht_hbm.at[base_col + j], hbufs[j], sem0))
            inits.append(
                pltpu.async_copy(ht_hbm.at[base_col + j], abufs[j], sem1))
        for d in inits:
            d.wait()
        pend = start(0)
        for ci in range(NCH):
            cp = pA if ci % 2 == 0 else pB
            pend.wait()
            if ci + 1 < NCH:
                pend_next = start(ci + 1)

            @plsc.parallel_loop(0, CH, step=16, unroll=4)
            def _groups(b, cp=cp):
                pk = cp[pl.ds(b, 16)]
                si = jnp.bitwise_and(pk, 16383)
                ti = jnp.right_shift(pk, 14)
                vs = [plsc.load_gather(hbufs[j], [si]) for j in range(CPP)]
                for j in range(CPP):
                    plsc.addupdate_scatter(abufs[j], [ti], vs[j])

            if ci + 1 < NCH:
                pend = pend_next
        for j in range(CPP):
            wb.append(pltpu.async_copy(abufs[j], acc_hbm.at[base_col + j],
                                       sem0))
            abuf = abufs[j]

            @pl.loop(0, NP // 16, init_carry=(zeros16, zeros16), unroll=4)
            def _sums(i, c, abuf=abuf):
                s1v, s2v = c
                t = abuf[pl.ds(i * 16, 16)] * dv[pl.ds(i * 16, 16)]
                return (s1v + t, s2v + t * t)

            s1v, s2v = _sums
            s1 = jnp.sum(s1v)
            s2 = jnp.sum(s2v)
            sbuf[...] = jnp.where(lane == 0, s1,
                                  jnp.where(lane == 1, s2, 0.0))
            pltpu.sync_copy(sbuf, stat_hbm.at[base_col + j])
    for d in wb:
        d.wait()


@functools.cache
def _agg_kernel():
    return pl.kernel(
        _agg_body,
        out_type=[jax.ShapeDtypeStruct((D, NP), jnp.float32),
                  jax.ShapeDtypeStruct((D, 16), jnp.float32)],
        mesh=_mesh(),
        compiler_params=_SC_PARAMS,
        scratch_types=[
            pltpu.VMEM((CH,), jnp.int32),
            pltpu.VMEM((CH,), jnp.int32),
            pltpu.VMEM((NP,), jnp.float32),
            pltpu.VMEM((NP,), jnp.float32),
            pltpu.VMEM((NP,), jnp.float32),
            pltpu.VMEM((NP,), jnp.float32),
            pltpu.VMEM((NP,), jnp.float32),
            pltpu.VMEM((NP,), jnp.float32),
            pltpu.VMEM((NP,), jnp.float32),
            pltpu.VMEM((NP,), jnp.float32),
            pltpu.VMEM((NP,), jnp.float32),
            pltpu.VMEM((16,), jnp.float32),
            pltpu.SemaphoreType.DMA,
            pltpu.SemaphoreType.DMA,
        ],
    )


def _prep_body(part_ref, x_ref, w1_ref, ht_ref, dinv_ref):
    deg = jnp.sum(part_ref[...], axis=0, keepdims=True) + 1.0
    dinv = lax.rsqrt(deg)
    dinv_ref[...] = dinv
    h = jnp.dot(x_ref[...], w1_ref[...],
                preferred_element_type=jnp.float32,
                precision=lax.Precision.HIGHEST)
    ht_ref[...] = h.T * dinv


_prep_kernel = pl.pallas_call(
    _prep_body,
    grid=(NGRID,),
    in_specs=[
        pl.BlockSpec((NW, NB), lambda i: (0, i)),
        pl.BlockSpec((NB, D), lambda i: (i, 0)),
        pl.BlockSpec((D, D), lambda i: (0, 0)),
    ],
    out_specs=[
        pl.BlockSpec((D, NB), lambda i: (0, i)),
        pl.BlockSpec((1, NB), lambda i: (0, i)),
    ],
    out_shape=[
        jax.ShapeDtypeStruct((D, NP), jnp.float32),
        jax.ShapeDtypeStruct((1, NP), jnp.float32),
    ],
)


def _bn_coeffs(s_ref, g_ref, be_ref):
    m = s_ref[:, 0:1] * (1.0 / N)
    v = s_ref[:, 1:2] * (1.0 / N) - m * m
    alpha = g_ref[...] * lax.rsqrt(v + EPS)
    beta = be_ref[...] - alpha * m
    return alpha, beta


def _mid_body(acc_ref, dinv_ref, s_ref, g_ref, be_ref, w2t_ref, mask_ref,
              ht2_ref):
    alpha, beta = _bn_coeffs(s_ref, g_ref, be_ref)
    pre = acc_ref[...] * dinv_ref[...]
    r = jnp.maximum(alpha * pre + beta, 0.0) * mask_ref[...]
    ht2_ref[...] = jnp.dot(w2t_ref[...], r,
                           preferred_element_type=jnp.float32,
                           precision=lax.Precision.HIGHEST) * dinv_ref[...]


_mid_kernel = pl.pallas_call(
    _mid_body,
    grid=(NGRID,),
    in_specs=[
        pl.BlockSpec((D, NB), lambda i: (0, i)),
        pl.BlockSpec((1, NB), lambda i: (0, i)),
        pl.BlockSpec((D, 16), lambda i: (0, 0)),
        pl.BlockSpec((D, 1), lambda i: (0, 0)),
        pl.BlockSpec((D, 1), lambda i: (0, 0)),
        pl.BlockSpec((D, D), lambda i: (0, 0)),
        pl.BlockSpec((1, NB), lambda i: (0, i)),
    ],
    out_specs=pl.BlockSpec((D, NB), lambda i: (0, i)),
    out_shape=jax.ShapeDtypeStruct((D, NP), jnp.float32),
)


def _final_body(acc_ref, dinv_ref, s_ref, g_ref, be_ref, x_ref, out_ref):
    alpha, beta = _bn_coeffs(s_ref, g_ref, be_ref)
    pre = acc_ref[...] * dinv_ref[...]
    r = jnp.maximum(alpha * pre + beta, 0.0)
    out_ref[...] = r.T + x_ref[...]


_final_kernel = pl.pallas_call(
    _final_body,
    grid=(NGRID,),
    in_specs=[
        pl.BlockSpec((D, NB), lambda i: (0, i)),
        pl.BlockSpec((1, NB), lambda i: (0, i)),
        pl.BlockSpec((D, 16), lambda i: (0, 0)),
        pl.BlockSpec((D, 1), lambda i: (0, 0)),
        pl.BlockSpec((D, 1), lambda i: (0, 0)),
        pl.BlockSpec((NB, D), lambda i: (i, 0)),
    ],
    out_specs=pl.BlockSpec((NB, D), lambda i: (i, 0)),
    out_shape=jax.ShapeDtypeStruct((NP, D), jnp.float32),
)


@jax.jit
def kernel(x, edge_index, W1, b1, g1, be1, W2, b2, g2, be2):
    del b1, b2
    xp = jnp.pad(x, ((0, NP - N), (0, 0)))
    src = edge_index[0]
    dst = edge_index[1]
    mask = (jnp.arange(NP, dtype=jnp.int32) < N).astype(jnp.float32)
    mask = mask.reshape(1, NP)

    part, packed = _deg_kernel()(src, dst)
    ht1, dinv = _prep_kernel(part, xp, W1)
    acc1, s1 = _agg_kernel()(ht1, packed, dinv)
    ht2 = _mid_kernel(acc1, dinv, s1, g1.reshape(D, 1), be1.reshape(D, 1),
                      W2.T, mask)
    acc2, s2 = _agg_kernel()(ht2, packed, dinv)
    outp = _final_kernel(acc2, dinv, s2, g2.reshape(D, 1), be2.reshape(D, 1),
                         xp)
    return outp[:N]
